# baseline (speedup 1.0000x reference)
import jax
import jax.numpy as jnp
from jax import lax
from jax.experimental import pallas as pl
from jax.experimental.pallas import tpu as pltpu
from jax.experimental.pallas import tpu_sc as plsc

N_CAT = 1000
BATCH = 16384
L = 16
NC, NS = 2, 16
NW = NC * NS
CPW = BATCH // NW
R0, R1 = 104, 96
NPAIR = N_CAT // (R0 + R1)
ZCOL = 384


def _body(obs_hbm, eye_hbm, out_hbm, idx_v, zb0, zb1, sem0, sem1):
    wid = lax.axis_index("s") * NC + lax.axis_index("c")
    colbase = wid * CPW
    pltpu.sync_copy(obs_hbm.at[pl.ds(colbase, CPW)], idx_v)

    pltpu.async_copy(eye_hbm.at[pl.ds(0, R0), pl.ds(ZCOL, CPW)], zb0, sem0)
    pltpu.async_copy(eye_hbm.at[pl.ds(0, R1), pl.ds(ZCOL, CPW)], zb1, sem1)

    zeros = jnp.zeros((L,), jnp.float32)
    ones = jnp.ones((L,), jnp.float32)
    iota = lax.broadcasted_iota(jnp.int32, (L,), 0)

    def clear_set_pass(zb, pr0, r0, nr):
        def one(j, _):
            v = idx_v[pl.ds(j * L, L)]
            col = iota + j * L
            pmask = (v >= pr0) & (v < pr0 + nr)
            plsc.store_scatter(zb, [v - pr0, col], zeros, mask=pmask)
            mask = (v >= r0) & (v < r0 + nr)
            plsc.store_scatter(zb, [v - r0, col], ones, mask=mask)
            return 0
        lax.fori_loop(0, CPW // L, one, 0, unroll=4)

    def pair(i, _):
        r0 = i * (R0 + R1)
        pltpu.make_async_copy(
            zb0, out_hbm.at[pl.ds(0, R0), pl.ds(colbase, CPW)], sem0).wait()
        clear_set_pass(zb0, r0 - (R0 + R1), r0, R0)
        pltpu.async_copy(
            zb0, out_hbm.at[pl.ds(r0, R0), pl.ds(colbase, CPW)], sem0)

        pltpu.make_async_copy(
            zb1, out_hbm.at[pl.ds(0, R1), pl.ds(colbase, CPW)], sem1).wait()
        clear_set_pass(zb1, r0 - R1, r0 + R0, R1)
        pltpu.async_copy(
            zb1, out_hbm.at[pl.ds(r0 + R0, R1), pl.ds(colbase, CPW)], sem1)
        return 0

    lax.fori_loop(0, NPAIR, pair, 0)
    pltpu.make_async_copy(
        zb0, out_hbm.at[pl.ds(0, R0), pl.ds(colbase, CPW)], sem0).wait()
    pltpu.make_async_copy(
        zb1, out_hbm.at[pl.ds(0, R1), pl.ds(colbase, CPW)], sem1).wait()


@jax.jit
def kernel(obs, eye):
    mesh = plsc.VectorSubcoreMesh(core_axis_name="c", subcore_axis_name="s")
    out_t = pl.kernel(
        _body,
        out_type=jax.ShapeDtypeStruct((N_CAT, BATCH), jnp.float32),
        mesh=mesh,
        compiler_params=pltpu.CompilerParams(
            needs_layout_passes=False, use_tc_tiling_on_sc=True),
        scratch_types=[
            pltpu.VMEM((CPW,), jnp.int32),
            pltpu.VMEM((R0, CPW), jnp.float32),
            pltpu.VMEM((R1, CPW), jnp.float32),
            pltpu.SemaphoreType.DMA,
            pltpu.SemaphoreType.DMA,
        ],
    )(obs, eye)
    return out_t.T


# speedup vs baseline: 1.4153x; 1.4153x over previous
import jax
import jax.numpy as jnp
from jax import lax
from jax.experimental import pallas as pl
from jax.experimental.pallas import tpu as pltpu
from jax.experimental.pallas import tpu_sc as plsc

N_CAT = 1000
BATCH = 16384
L = 16
NC, NS = 2, 16
NW = NC * NS
CPW = BATCH // NW
R0, R1 = 104, 96
NPAIR = N_CAT // (R0 + R1)


def _body(obs_hbm, eye_hbm, out_hbm, idx_v, zb0, zb1, sem0, sem1):
    del eye_hbm
    wid = lax.axis_index("s") * NC + lax.axis_index("c")
    colbase = wid * CPW
    pltpu.sync_copy(obs_hbm.at[pl.ds(colbase, CPW)], idx_v)

    zeros = jnp.zeros((L,), jnp.float32)
    ones = jnp.ones((L,), jnp.float32)
    iota = lax.broadcasted_iota(jnp.int32, (L,), 0)

    def zinit(zb):
        def step(i, _):
            zb[i // 32, pl.ds((i % 32) * L, L)] = zeros
            return 0
        return step

    def clear_set_pass(zb, pr0, r0, nr):
        def one(j, _):
            v = idx_v[pl.ds(j * L, L)]
            col = iota + j * L
            pmask = (v >= pr0) & (v < pr0 + nr)
            plsc.store_scatter(zb, [v - pr0, col], zeros, mask=pmask)
            mask = (v >= r0) & (v < r0 + nr)
            plsc.store_scatter(zb, [v - r0, col], ones, mask=mask)
            return 0
        lax.fori_loop(0, CPW // L, one, 0, unroll=4)

    lax.fori_loop(0, R0 * 32, zinit(zb0), 0, unroll=8)

    def pair(i, _):
        r0 = i * (R0 + R1)
        @pl.when(i > 0)
        def _():
            pltpu.make_async_copy(
                zb0, out_hbm.at[pl.ds(0, R0), pl.ds(colbase, CPW)], sem0).wait()

        clear_set_pass(zb0, r0 - (R0 + R1), r0, R0)
        pltpu.async_copy(
            zb0, out_hbm.at[pl.ds(r0, R0), pl.ds(colbase, CPW)], sem0)

        @pl.when(i == 0)
        def _():
            lax.fori_loop(0, R1 * 32, zinit(zb1), 0, unroll=8)

        @pl.when(i > 0)
        def _():
            pltpu.make_async_copy(
                zb1, out_hbm.at[pl.ds(0, R1), pl.ds(colbase, CPW)], sem1).wait()

        clear_set_pass(zb1, r0 - R1, r0 + R0, R1)
        pltpu.async_copy(
            zb1, out_hbm.at[pl.ds(r0 + R0, R1), pl.ds(colbase, CPW)], sem1)
        return 0

    lax.fori_loop(0, NPAIR, pair, 0)
    pltpu.make_async_copy(
        zb0, out_hbm.at[pl.ds(0, R0), pl.ds(colbase, CPW)], sem0).wait()
    pltpu.make_async_copy(
        zb1, out_hbm.at[pl.ds(0, R1), pl.ds(colbase, CPW)], sem1).wait()


@jax.jit
def kernel(obs, eye):
    mesh = plsc.VectorSubcoreMesh(core_axis_name="c", subcore_axis_name="s")
    out_t = pl.kernel(
        _body,
        out_type=jax.ShapeDtypeStruct((N_CAT, BATCH), jnp.float32),
        mesh=mesh,
        compiler_params=pltpu.CompilerParams(
            needs_layout_passes=False, use_tc_tiling_on_sc=True),
        scratch_types=[
            pltpu.VMEM((CPW,), jnp.int32),
            pltpu.VMEM((R0, CPW), jnp.float32),
            pltpu.VMEM((R1, CPW), jnp.float32),
            pltpu.SemaphoreType.DMA,
            pltpu.SemaphoreType.DMA,
        ],
    )(obs, eye)
    return out_t.T
